# baseline (speedup 1.0000x reference)
import functools

import jax
import jax.numpy as jnp
from jax.experimental import pallas as pl


def _ffn_body(x_ref, tw_ref, ti_ref, ga_ref, gb_ref, ua_ref, ub_ref,
              w2a_ref, w2b_ref, w2c_ref, w2d_ref, out_ref):
    e = pl.program_id(0)
    F2 = ga_ref.shape[2]
    Dq = w2a_ref.shape[1]

    @pl.when(e == 0)
    def _():
        out_ref[...] = jnp.zeros_like(out_ref)

    x = x_ref[...]

    def dott(a, b):
        return jax.lax.dot_general(a, b, (((1,), (1,)), ((), ())),
                                   preferred_element_type=jnp.float32)

    g1 = dott(x, ga_ref[0, 0])
    u1 = dott(x, ua_ref[0, 0])
    act1 = (g1 * jax.nn.sigmoid(g1)) * u1
    g2 = dott(x, gb_ref[0, 0])
    u2 = dott(x, ub_ref[0, 0])
    act2 = (g2 * jax.nn.sigmoid(g2)) * u2

    ids = ti_ref[...]
    tw = tw_ref[...]
    wvec = jnp.sum(jnp.where(ids == e, tw, 0.0), axis=1)[:, None]

    for q, w2q in enumerate((w2a_ref, w2b_ref, w2c_ref, w2d_ref)):
        wq = w2q[0]
        yq = dott(act1, wq[:, :F2]) + dott(act2, wq[:, F2:])
        out_ref[:, q * Dq:(q + 1) * Dq] += wvec * yq


@jax.jit
def kernel(hidden_states, topk_weights, topk_ids, w1, w2):
    T, D = hidden_states.shape
    E = w1.shape[0]
    F = w1.shape[1] // 2

    w1r = w1.reshape(E, 4, F // 2, D)

    grid = (E,)
    w1spec = lambda q: pl.BlockSpec((1, 1, F // 2, D), lambda e, q=q: (e, q, 0, 0))
    w2spec = lambda q: pl.BlockSpec((1, D // 4, F), lambda e, q=q: (e, q, 0))
    out = pl.pallas_call(
        _ffn_body,
        grid=grid,
        in_specs=[
            pl.BlockSpec((T, D), lambda e: (0, 0)),
            pl.BlockSpec(topk_weights.shape, lambda e: (0, 0)),
            pl.BlockSpec(topk_ids.shape, lambda e: (0, 0)),
            w1spec(0), w1spec(1), w1spec(2), w1spec(3),
            w2spec(0), w2spec(1), w2spec(2), w2spec(3),
        ],
        out_specs=pl.BlockSpec((T, D), lambda e: (0, 0)),
        out_shape=jax.ShapeDtypeStruct((T, D), jnp.float32),
    )(hidden_states, topk_weights, topk_ids,
      w1r, w1r, w1r, w1r, w2, w2, w2, w2)
    return out


# speedup vs baseline: 1.0040x; 1.0040x over previous
import functools

import jax
import jax.numpy as jnp
from jax.experimental import pallas as pl


def _ffn_body(x_ref, tw_ref, ti_ref, wg_ref, wu_ref,
              w2a_ref, w2b_ref, w2c_ref, w2d_ref, out_ref):
    e = pl.program_id(0)
    Dq = w2a_ref.shape[1]

    @pl.when(e == 0)
    def _():
        out_ref[...] = jnp.zeros_like(out_ref)

    x = x_ref[...]

    def dott(a, b):
        return jax.lax.dot_general(a, b, (((1,), (1,)), ((), ())),
                                   preferred_element_type=jnp.float32)

    g = dott(x, wg_ref[0, 0])
    u = dott(x, wu_ref[0, 0])
    act = (g * jax.nn.sigmoid(g)) * u

    ids = ti_ref[...]
    tw = tw_ref[...]
    wvec = jnp.sum(jnp.where(ids == e, tw, 0.0), axis=1)[:, None]

    for q, w2q in enumerate((w2a_ref, w2b_ref, w2c_ref, w2d_ref)):
        yq = dott(act, w2q[0])
        out_ref[:, q * Dq:(q + 1) * Dq] += wvec * yq


@jax.jit
def kernel(hidden_states, topk_weights, topk_ids, w1, w2):
    T, D = hidden_states.shape
    E = w1.shape[0]
    F = w1.shape[1] // 2

    w1r = w1.reshape(E, 2, F, D)

    grid = (E,)
    w1spec = lambda q: pl.BlockSpec((1, 1, F, D), lambda e, q=q: (e, q, 0, 0))
    w2spec = lambda q: pl.BlockSpec((1, D // 4, F), lambda e, q=q: (e, q, 0))
    out = pl.pallas_call(
        _ffn_body,
        grid=grid,
        in_specs=[
            pl.BlockSpec((T, D), lambda e: (0, 0)),
            pl.BlockSpec(topk_weights.shape, lambda e: (0, 0)),
            pl.BlockSpec(topk_ids.shape, lambda e: (0, 0)),
            w1spec(0), w1spec(1),
            w2spec(0), w2spec(1), w2spec(2), w2spec(3),
        ],
        out_specs=pl.BlockSpec((T, D), lambda e: (0, 0)),
        out_shape=jax.ShapeDtypeStruct((T, D), jnp.float32),
    )(hidden_states, topk_weights, topk_ids,
      w1r, w1r, w2, w2, w2, w2)
    return out
